# baseline (speedup 1.0000x reference)
import functools

import jax
import jax.numpy as jnp
from jax import lax
from jax.experimental import pallas as pl
from jax.experimental.pallas import tpu as pltpu
from jax.experimental.pallas import tpu_sc as plsc

N = 10000
D = 128
NUM_CORES = 2
NUM_SUBCORES = 16
NW = NUM_CORES * NUM_SUBCORES
CHUNK = 128
CH = 80
EPT = CH * CHUNK
EP = NW * EPT
NPAD = 10112
RPS = NPAD // NUM_SUBCORES

_mesh = plsc.VectorSubcoreMesh(core_axis_name="c", subcore_axis_name="s")


def _scatter_add_layer(h, idx_g, zeros):
    @functools.partial(
        pl.kernel,
        out_type=jax.ShapeDtypeStruct((NUM_CORES, NPAD, D), jnp.float32),
        mesh=_mesh,
        scratch_types=[
            pltpu.VMEM((2, CHUNK), jnp.int32),
            pltpu.VMEM((2, CHUNK), jnp.int32),
            pltpu.VMEM((CHUNK, D), jnp.float32),
            pltpu.VMEM((CHUNK, D), jnp.float32),
            pltpu.VMEM_SHARED((NPAD, D), jnp.float32),
            pltpu.SemaphoreType.DMA,
            pltpu.SemaphoreType.DMA,
            pltpu.SemaphoreType.DMA,
            pltpu.SemaphoreType.DMA,
            pltpu.SemaphoreType.DMA,
            pltpu.SemaphoreType.DMA,
        ],
    )
    def k(h_hbm, idx_hbm, z_hbm, out_hbm,
          ib_a, ib_b, db_a, db_b, acc,
          sem_ia, sem_ib, sem_ga, sem_gb, sem_sa, sem_sb):
        c = lax.axis_index("c")
        s = lax.axis_index("s")
        wid = c * NUM_SUBCORES + s
        pltpu.sync_copy(z_hbm, acc.at[pl.ds(s * RPS, RPS)])
        pltpu.async_copy(idx_hbm.at[wid, 0], ib_a, sem_ia)
        pltpu.async_copy(idx_hbm.at[wid, 1], ib_b, sem_ib)
        plsc.subcore_barrier()

        @pl.loop(0, CH, step=2)
        def _(j):
            pltpu.make_async_copy(idx_hbm.at[wid, j], ib_a, sem_ia).wait()
            cg_a1 = pltpu.async_copy(h_hbm.at[ib_a.at[0, pl.ds(0, 64)]],
                                     db_a.at[pl.ds(0, 64)], sem_ga)
            cg_a2 = pltpu.async_copy(h_hbm.at[ib_a.at[0, pl.ds(64, 64)]],
                                     db_a.at[pl.ds(64, 64)], sem_ga)
            pltpu.make_async_copy(idx_hbm.at[wid, j + 1], ib_b, sem_ib).wait()
            cg_b1 = pltpu.async_copy(h_hbm.at[ib_b.at[0, pl.ds(0, 64)]],
                                     db_b.at[pl.ds(0, 64)], sem_gb)
            cg_b2 = pltpu.async_copy(h_hbm.at[ib_b.at[0, pl.ds(64, 64)]],
                                     db_b.at[pl.ds(64, 64)], sem_gb)
            cg_a1.wait()
            cg_a2.wait()
            cs_a = pltpu.async_copy(db_a, acc.at[ib_a.at[1]], sem_sa, add=True)
            cg_b1.wait()
            cg_b2.wait()
            cs_b = pltpu.async_copy(db_b, acc.at[ib_b.at[1]], sem_sb, add=True)
            cs_a.wait()

            @pl.when(j + 2 < CH)
            def _():
                pltpu.async_copy(idx_hbm.at[wid, j + 2], ib_a, sem_ia)

            cs_b.wait()

            @pl.when(j + 3 < CH)
            def _():
                pltpu.async_copy(idx_hbm.at[wid, j + 3], ib_b, sem_ib)

        plsc.subcore_barrier()
        pltpu.sync_copy(acc.at[pl.ds(s * RPS, RPS)],
                        out_hbm.at[c, pl.ds(s * RPS, RPS)])

    return k(h, idx_g, zeros)


def _mm_body(x_ref, w_ref, o_ref):
    o_ref[...] = lax.dot_general(
        x_ref[...], w_ref[...], (((1,), (0,)), ((), ())),
        precision=lax.Precision.HIGHEST, preferred_element_type=jnp.float32)


def _matmul(x, w, block):
    m = x.shape[0]
    return pl.pallas_call(
        _mm_body,
        grid=(m // block,),
        in_specs=[pl.BlockSpec((block, D), lambda i: (i, 0)),
                  pl.BlockSpec((D, D), lambda i: (0, 0))],
        out_specs=pl.BlockSpec((block, D), lambda i: (i, 0)),
        out_shape=jax.ShapeDtypeStruct((m, D), jnp.float32),
    )(x, w)


def _mid_body(p_ref, b_ref, w_ref, o_ref):
    z = jnp.maximum(p_ref[0] + p_ref[1] + b_ref[...], 0.0)
    o_ref[...] = lax.dot_general(
        z, w_ref[...], (((1,), (0,)), ((), ())),
        precision=lax.Precision.HIGHEST, preferred_element_type=jnp.float32)


def _mid(partials, b, w, block):
    return pl.pallas_call(
        _mid_body,
        grid=(N // block,),
        in_specs=[pl.BlockSpec((NUM_CORES, block, D), lambda i: (0, i, 0)),
                  pl.BlockSpec((1, D), lambda i: (0, 0)),
                  pl.BlockSpec((D, D), lambda i: (0, 0))],
        out_specs=pl.BlockSpec((block, D), lambda i: (i, 0)),
        out_shape=jax.ShapeDtypeStruct((N, D), jnp.float32),
    )(partials, b, w)


def _final_body(q_ref, b_ref, o_ref):
    z = q_ref[0] + q_ref[1] + b_ref[...]
    m = jnp.max(z, axis=-1, keepdims=True)
    e = z - m
    lse = jnp.log(jnp.sum(jnp.exp(e), axis=-1, keepdims=True))
    o_ref[...] = e - lse


def _final(partials, b, block):
    return pl.pallas_call(
        _final_body,
        grid=(N // block,),
        in_specs=[pl.BlockSpec((NUM_CORES, block, D), lambda i: (0, i, 0)),
                  pl.BlockSpec((1, D), lambda i: (0, 0))],
        out_specs=pl.BlockSpec((block, D), lambda i: (i, 0)),
        out_shape=jax.ShapeDtypeStruct((N, D), jnp.float32),
    )(partials, b)


def _prep_edges(edge_index):
    src = edge_index[0].astype(jnp.int32)
    dst = edge_index[1].astype(jnp.int32)
    pad = EP - src.shape[0]
    src = jnp.concatenate([src, jnp.zeros((pad,), jnp.int32)])
    dst = jnp.concatenate([dst, jnp.full((pad,), NPAD - 1, jnp.int32)])
    return jnp.stack([src.reshape(NW, CH, CHUNK),
                      dst.reshape(NW, CH, CHUNK)], axis=2)


def kernel(x, edge_index1, edge_index2, W1, b1, W2, b2):
    idx1 = _prep_edges(edge_index1)
    idx2 = _prep_edges(edge_index2)
    zeros = jnp.zeros((RPS, D), jnp.float32)
    h1 = _matmul(x, W1, 2000)
    p1 = _scatter_add_layer(h1, idx1, zeros)
    h2 = _mid(p1, b1.reshape(1, D), W2, 2000)
    p2 = _scatter_add_layer(h2, idx2, zeros)
    return _final(p2, b2.reshape(1, D), 2000)


# speedup vs baseline: 1.0110x; 1.0110x over previous
import functools

import jax
import jax.numpy as jnp
from jax import lax
from jax.experimental import pallas as pl
from jax.experimental.pallas import tpu as pltpu
from jax.experimental.pallas import tpu_sc as plsc

N = 10000
D = 128
NUM_CORES = 2
NUM_SUBCORES = 16
NW = NUM_CORES * NUM_SUBCORES
CHUNK = 128
CH = 80
EPT = CH * CHUNK
EP = NW * EPT
NPAD = 10112
RPS = NPAD // NUM_SUBCORES

_mesh = plsc.VectorSubcoreMesh(core_axis_name="c", subcore_axis_name="s")


def _scatter_add_layer(h, idx_g, zeros):
    @functools.partial(
        pl.kernel,
        out_type=jax.ShapeDtypeStruct((NUM_CORES, NPAD, D), jnp.float32),
        mesh=_mesh,
        scratch_types=[
            pltpu.VMEM((2, CHUNK), jnp.int32),
            pltpu.VMEM((2, CHUNK), jnp.int32),
            pltpu.VMEM((CHUNK, D), jnp.float32),
            pltpu.VMEM((CHUNK, D), jnp.float32),
            pltpu.VMEM_SHARED((NPAD, D), jnp.float32),
            pltpu.SemaphoreType.DMA,
            pltpu.SemaphoreType.DMA,
            pltpu.SemaphoreType.DMA,
            pltpu.SemaphoreType.DMA,
            pltpu.SemaphoreType.DMA,
            pltpu.SemaphoreType.DMA,
        ],
    )
    def k(h_hbm, idx_hbm, z_hbm, out_hbm,
          ib_a, ib_b, db_a, db_b, acc,
          sem_ia, sem_ib, sem_ga, sem_gb, sem_sa, sem_sb):
        c = lax.axis_index("c")
        s = lax.axis_index("s")
        wid = c * NUM_SUBCORES + s
        pltpu.sync_copy(z_hbm, acc.at[pl.ds(s * RPS, RPS)])
        pltpu.async_copy(idx_hbm.at[wid, 0], ib_a, sem_ia)
        pltpu.async_copy(idx_hbm.at[wid, 1], ib_b, sem_ib)
        plsc.subcore_barrier()

        @pl.loop(0, CH, step=2)
        def _(j):
            pltpu.make_async_copy(idx_hbm.at[wid, j], ib_a, sem_ia).wait()
            cg_a = pltpu.async_copy(h_hbm.at[ib_a.at[0]], db_a, sem_ga)
            pltpu.make_async_copy(idx_hbm.at[wid, j + 1], ib_b, sem_ib).wait()
            cg_b = pltpu.async_copy(h_hbm.at[ib_b.at[0]], db_b, sem_gb)
            cg_a.wait()
            cs_a = pltpu.async_copy(db_a, acc.at[ib_a.at[1]], sem_sa, add=True)
            cg_b.wait()
            cs_b = pltpu.async_copy(db_b, acc.at[ib_b.at[1]], sem_sb, add=True)
            cs_a.wait()

            @pl.when(j + 2 < CH)
            def _():
                pltpu.async_copy(idx_hbm.at[wid, j + 2], ib_a, sem_ia)

            cs_b.wait()

            @pl.when(j + 3 < CH)
            def _():
                pltpu.async_copy(idx_hbm.at[wid, j + 3], ib_b, sem_ib)

        plsc.subcore_barrier()
        pltpu.sync_copy(acc.at[pl.ds(s * RPS, RPS)],
                        out_hbm.at[c, pl.ds(s * RPS, RPS)])

    return k(h, idx_g, zeros)


def _mm_body(x_ref, w_ref, o_ref):
    o_ref[...] = lax.dot_general(
        x_ref[...], w_ref[...], (((1,), (0,)), ((), ())),
        precision=lax.Precision.HIGHEST, preferred_element_type=jnp.float32)


def _matmul(x, w, block):
    m = x.shape[0]
    return pl.pallas_call(
        _mm_body,
        grid=(m // block,),
        in_specs=[pl.BlockSpec((block, D), lambda i: (i, 0)),
                  pl.BlockSpec((D, D), lambda i: (0, 0))],
        out_specs=pl.BlockSpec((block, D), lambda i: (i, 0)),
        out_shape=jax.ShapeDtypeStruct((m, D), jnp.float32),
    )(x, w)


def _mid_body(p_ref, b_ref, w_ref, o_ref):
    z = jnp.maximum(p_ref[0] + p_ref[1] + b_ref[...], 0.0)
    o_ref[...] = lax.dot_general(
        z, w_ref[...], (((1,), (0,)), ((), ())),
        precision=lax.Precision.HIGHEST, preferred_element_type=jnp.float32)


def _mid(partials, b, w, block):
    return pl.pallas_call(
        _mid_body,
        grid=(N // block,),
        in_specs=[pl.BlockSpec((NUM_CORES, block, D), lambda i: (0, i, 0)),
                  pl.BlockSpec((1, D), lambda i: (0, 0)),
                  pl.BlockSpec((D, D), lambda i: (0, 0))],
        out_specs=pl.BlockSpec((block, D), lambda i: (i, 0)),
        out_shape=jax.ShapeDtypeStruct((N, D), jnp.float32),
    )(partials, b, w)


def _final_body(q_ref, b_ref, o_ref):
    z = q_ref[0] + q_ref[1] + b_ref[...]
    m = jnp.max(z, axis=-1, keepdims=True)
    e = z - m
    lse = jnp.log(jnp.sum(jnp.exp(e), axis=-1, keepdims=True))
    o_ref[...] = e - lse


def _final(partials, b, block):
    return pl.pallas_call(
        _final_body,
        grid=(N // block,),
        in_specs=[pl.BlockSpec((NUM_CORES, block, D), lambda i: (0, i, 0)),
                  pl.BlockSpec((1, D), lambda i: (0, 0))],
        out_specs=pl.BlockSpec((block, D), lambda i: (i, 0)),
        out_shape=jax.ShapeDtypeStruct((N, D), jnp.float32),
    )(partials, b)


def _prep_edges(edge_index):
    src = edge_index[0].astype(jnp.int32)
    dst = edge_index[1].astype(jnp.int32)
    pad = EP - src.shape[0]
    src = jnp.concatenate([src, jnp.zeros((pad,), jnp.int32)])
    dst = jnp.concatenate([dst, jnp.full((pad,), NPAD - 1, jnp.int32)])
    return jnp.stack([src.reshape(NW, CH, CHUNK),
                      dst.reshape(NW, CH, CHUNK)], axis=2)


def kernel(x, edge_index1, edge_index2, W1, b1, W2, b2):
    idx1 = _prep_edges(edge_index1)
    idx2 = _prep_edges(edge_index2)
    zeros = jnp.zeros((RPS, D), jnp.float32)
    h1 = _matmul(x, W1, 2000)
    p1 = _scatter_add_layer(h1, idx1, zeros)
    h2 = _mid(p1, b1.reshape(1, D), W2, 2000)
    p2 = _scatter_add_layer(h2, idx2, zeros)
    return _final(p2, b2.reshape(1, D), 2000)
